# baseline (speedup 1.0000x reference)
import jax
import jax.numpy as jnp
from jax.experimental import pallas as pl
from jax.experimental.pallas import tpu as pltpu

_NE = 1024
_ED = 64
_CC = 0.25
_B = 4
_S = 8192
_BS = 2048
_NBLK = _S // _BS
_GRID = _B * _NBLK
_NTOK = _B * _S


def _vq_kernel(x_ref, e_ref, et_ref,
               oh_ref, q_ref, idx_ref, cnt_ref, loss_ref, perp_ref):
    g = pl.program_id(0)

    @pl.when(g == 0)
    def _init():
        cnt_ref[...] = jnp.zeros_like(cnt_ref)
        loss_ref[...] = jnp.zeros_like(loss_ref)

    x = x_ref[0]
    e = e_ref[...]

    s = jnp.dot(e, x, preferred_element_type=jnp.float32)
    xsq = jnp.sum(x * x, axis=0, keepdims=True)
    esq = jnp.sum(e * e, axis=1, keepdims=True)
    dist = xsq + esq - 2.0 * s

    kiota = jax.lax.broadcasted_iota(jnp.int32, (_NE, _BS), 0)
    dmin = jnp.min(dist, axis=0, keepdims=True)
    idx = jnp.min(jnp.where(dist == dmin, kiota, _NE), axis=0)
    idx_ref[0, 0] = idx

    oh = (kiota == idx[None, :]).astype(jnp.float32)
    oh_ref[0] = oh

    q = jax.lax.dot_general(et_ref[...], oh, (((1,), (0,)), ((), ())),
                            precision=jax.lax.Precision.HIGHEST,
                            preferred_element_type=jnp.float32)
    q_ref[0] = q

    cnt_ref[...] += jnp.sum(oh, axis=1, keepdims=True)
    loss_ref[...] += jnp.sum((q - x) ** 2, keepdims=True)

    @pl.when(g == _GRID - 1)
    def _fin():
        p = cnt_ref[...] * (1.0 / _NTOK)
        perp_ref[...] = jnp.exp(-jnp.sum(p * jnp.log(p + 1e-10),
                                         keepdims=True))
        loss_ref[...] = loss_ref[...] * (_CC / (_NTOK * _ED))


def kernel(inputs, embed):
    x = inputs.reshape(_B, _ED, _S)
    et = embed.T

    oh, q, idx, _cnt, loss, perp = pl.pallas_call(
        _vq_kernel,
        grid=(_GRID,),
        in_specs=[
            pl.BlockSpec((1, _ED, _BS), lambda g: (g // _NBLK, 0, g % _NBLK)),
            pl.BlockSpec((_NE, _ED), lambda g: (0, 0)),
            pl.BlockSpec((_ED, _NE), lambda g: (0, 0)),
        ],
        out_specs=[
            pl.BlockSpec((1, _NE, _BS), lambda g: (g // _NBLK, 0, g % _NBLK)),
            pl.BlockSpec((1, _ED, _BS), lambda g: (g // _NBLK, 0, g % _NBLK)),
            pl.BlockSpec((1, 1, _BS), lambda g: (g, 0, 0)),
            pl.BlockSpec((_NE, 1), lambda g: (0, 0)),
            pl.BlockSpec((1, 1), lambda g: (0, 0)),
            pl.BlockSpec((1, 1), lambda g: (0, 0)),
        ],
        out_shape=[
            jax.ShapeDtypeStruct((_B, _NE, _S), jnp.float32),
            jax.ShapeDtypeStruct((_B, _ED, _S), jnp.float32),
            jax.ShapeDtypeStruct((_GRID, 1, _BS), jnp.int32),
            jax.ShapeDtypeStruct((_NE, 1), jnp.float32),
            jax.ShapeDtypeStruct((1, 1), jnp.float32),
            jax.ShapeDtypeStruct((1, 1), jnp.float32),
        ],
        compiler_params=pltpu.CompilerParams(
            dimension_semantics=("arbitrary",),
        ),
    )(x, embed, et)

    quantized_st = q.reshape(_B, _ED, 8, 32, 32)
    oh_r = oh.reshape(_B, _NE, 8, 32, 32)
    encoding_indices = idx.reshape(_NTOK)
    return (loss[0, 0], quantized_st, perp[0, 0], oh_r, encoding_indices)


# speedup vs baseline: 1.2864x; 1.2864x over previous
import jax
import jax.numpy as jnp
from jax.experimental import pallas as pl
from jax.experimental.pallas import tpu as pltpu

_NE = 1024
_ED = 64
_CC = 0.25
_B = 4
_S = 8192
_BS = 2048
_NBLK = _S // _BS
_GRID = _B * _NBLK
_NTOK = _B * _S


def _vq_kernel(x_ref, e_ref, et_ref, oh_ref, q_ref, idx_ref, cnt_ref, lp_ref):
    x = x_ref[0]
    e = e_ref[...]

    s = jnp.dot(e, x, preferred_element_type=jnp.float32)
    xsq = jnp.sum(x * x, axis=0, keepdims=True)
    esq = jnp.sum(e * e, axis=1, keepdims=True)
    dist = xsq + esq - 2.0 * s

    kiota = jax.lax.broadcasted_iota(jnp.int32, (_NE, _BS), 0)
    dmin = jnp.min(dist, axis=0, keepdims=True)
    idx = jnp.min(jnp.where(dist == dmin, kiota, _NE), axis=0)
    idx_ref[0, 0] = idx

    oh = (kiota == idx[None, :]).astype(jnp.float32)
    oh_ref[0] = oh

    q = jnp.dot(et_ref[...], oh, preferred_element_type=jnp.float32)
    q_ref[0] = q

    cnt_ref[0, 0] = jnp.sum(oh, axis=1)
    lp_ref[0, 0] = jnp.broadcast_to(jnp.sum(dmin, axis=1), (_NE,))


def _fin_kernel(cnt_ref, lp_ref, loss_ref, perp_ref):
    cnt = jnp.sum(cnt_ref[...], axis=0)
    p = cnt * (1.0 / _NTOK)
    perp_ref[...] = jnp.exp(-jnp.sum(p * jnp.log(p + 1e-10), keepdims=True))
    lsum = jnp.sum(lp_ref[...][:, :, 0], keepdims=True)
    loss_ref[...] = lsum * (_CC / (_NTOK * _ED))


def kernel(inputs, embed):
    x = inputs.reshape(_B, _ED, _S)
    et = embed.T

    oh, q, idx, cnt, lp = pl.pallas_call(
        _vq_kernel,
        grid=(_GRID,),
        in_specs=[
            pl.BlockSpec((1, _ED, _BS), lambda g: (g // _NBLK, 0, g % _NBLK)),
            pl.BlockSpec((_NE, _ED), lambda g: (0, 0)),
            pl.BlockSpec((_ED, _NE), lambda g: (0, 0)),
        ],
        out_specs=[
            pl.BlockSpec((1, _NE, _BS), lambda g: (g // _NBLK, 0, g % _NBLK)),
            pl.BlockSpec((1, _ED, _BS), lambda g: (g // _NBLK, 0, g % _NBLK)),
            pl.BlockSpec((1, 1, _BS), lambda g: (g, 0, 0)),
            pl.BlockSpec((1, 1, _NE), lambda g: (g, 0, 0)),
            pl.BlockSpec((1, 1, _NE), lambda g: (g, 0, 0)),
        ],
        out_shape=[
            jax.ShapeDtypeStruct((_B, _NE, _S), jnp.float32),
            jax.ShapeDtypeStruct((_B, _ED, _S), jnp.float32),
            jax.ShapeDtypeStruct((_GRID, 1, _BS), jnp.int32),
            jax.ShapeDtypeStruct((_GRID, 1, _NE), jnp.float32),
            jax.ShapeDtypeStruct((_GRID, 1, _NE), jnp.float32),
        ],
        compiler_params=pltpu.CompilerParams(
            dimension_semantics=("parallel",),
        ),
    )(x, embed, et)

    loss, perp = pl.pallas_call(
        _fin_kernel,
        out_specs=[
            pl.BlockSpec((1, 1), lambda: (0, 0)),
            pl.BlockSpec((1, 1), lambda: (0, 0)),
        ],
        out_shape=[
            jax.ShapeDtypeStruct((1, 1), jnp.float32),
            jax.ShapeDtypeStruct((1, 1), jnp.float32),
        ],
    )(cnt, lp)

    quantized_st = q.reshape(_B, _ED, 8, 32, 32)
    oh_r = oh.reshape(_B, _NE, 8, 32, 32)
    encoding_indices = idx.reshape(_NTOK)
    return (loss[0, 0], quantized_st, perp[0, 0], oh_r, encoding_indices)
